# Initial kernel scaffold; baseline (speedup 1.0000x reference)
import jax
import jax.numpy as jnp
from jax.experimental import pallas as pl


def kernel(token_sequences, embedding, positional_embedding):
    raise NotImplementedError("write your pallas kernel here")


# speedup vs baseline: 2.8923x; 2.8923x over previous
import jax
import jax.numpy as jnp
from jax import lax
from jax.experimental import pallas as pl
from jax.experimental.pallas import tpu as pltpu, tpu_sc as plsc

EMB = 64
SCALE = 8.0
NC = 2
NS = 16
NW = NC * NS


def _make_sc_embed(n_seq: int, seq_len: int, vocab: int):
    assert n_seq % NW == 0
    seq_per_w = n_seq // NW
    h0 = 104
    h1 = seq_len - h0
    mesh = plsc.VectorSubcoreMesh(
        core_axis_name="c", subcore_axis_name="s",
        num_cores=NC, num_subcores=NS,
    )

    @pl.kernel(
        out_type=jax.ShapeDtypeStruct((n_seq * seq_len, EMB), jnp.float32),
        mesh=mesh,
        scratch_types=[
            pltpu.VMEM((2, h0), jnp.int32),
            pltpu.VMEM((2, h1), jnp.int32),
            pltpu.VMEM((2, seq_len, EMB), jnp.float32),
            pltpu.VMEM((seq_len, EMB), jnp.float32),
            pltpu.SemaphoreType.DMA,
        ],
        compiler_params=pltpu.CompilerParams(use_tc_tiling_on_sc=False),
    )
    def sc_embed(tok_hbm, pe_hbm, emb_hbm, out_hbm, idx0_v, idx1_v, rows_v,
                 pe_v, sem):
        wid = lax.axis_index("s") * NC + lax.axis_index("c")
        pltpu.sync_copy(pe_hbm, pe_v)

        @pl.loop(0, seq_per_w)
        def _seq(i):
            base = (wid * seq_per_w + i) * seq_len
            pltpu.sync_copy(tok_hbm.at[pl.ds(base, h0)], idx0_v.at[0])
            pltpu.sync_copy(tok_hbm.at[pl.ds(base + h0, h1)], idx1_v.at[0])
            cp0 = pltpu.async_copy(
                emb_hbm.at[idx0_v.at[0]], rows_v.at[0, pl.ds(0, h0)], sem)
            cp1 = pltpu.async_copy(
                emb_hbm.at[idx1_v.at[0]], rows_v.at[0, pl.ds(h0, h1)], sem)
            cp0.wait()
            cp1.wait()

            @pl.loop(0, seq_len)
            def _tok(t):
                for c in range(EMB // 16):
                    sl = pl.ds(c * 16, 16)
                    rows_v[0, t, sl] = rows_v[0, t, sl] * SCALE + pe_v[t, sl]

            pltpu.sync_copy(rows_v.at[0], out_hbm.at[pl.ds(base, seq_len)])

    return sc_embed


def kernel(token_sequences, embedding, positional_embedding):
    n_seq, seq_len = token_sequences.shape
    tok = token_sequences.reshape(-1).astype(jnp.int32)
    pe = positional_embedding[0, :seq_len, :]
    f = _make_sc_embed(n_seq, seq_len, embedding.shape[0])
    out = f(tok, pe, embedding)
    return out.reshape(n_seq, seq_len, EMB)
